# baseline (speedup 1.0000x reference)
import functools

import jax
import jax.numpy as jnp
from jax import lax
from jax.experimental import pallas as pl
from jax.experimental.pallas import tpu as pltpu
from jax.experimental.pallas import tpu_sc as plsc

N = 10000
E = 320000
D = 128
H = 64
R = 100
B = 100
OUT = 2

NC = 2
NS = 16
NW = NC * NS
EPW = E // NW
CH = 80
NCHUNK = EPW // CH
EROWS = E // CH
NPT = N // NS

_HIGHEST = lax.Precision.HIGHEST


def _sc_mesh():
    return plsc.VectorSubcoreMesh(core_axis_name="c", subcore_axis_name="s",
                                  num_cores=NC, num_subcores=NS)


@functools.cache
def _build_deg_kernel():
    return pl.kernel(
        _deg_body,
        out_type=jax.ShapeDtypeStruct((NW, N), jnp.float32),
        mesh=_sc_mesh(),
        scratch_types=[
            pltpu.VMEM((EPW,), jnp.int32),
            pltpu.VMEM((N,), jnp.float32),
        ],
        compiler_params=pltpu.CompilerParams(needs_layout_passes=False),
    )


def _deg_body(dst_hbm, out_hbm, didx, hist):
    c = lax.axis_index("c")
    s = lax.axis_index("s")
    wid = c * NS + s
    pltpu.sync_copy(dst_hbm.at[pl.ds(wid * EPW, EPW)], didx)
    zero16 = jnp.zeros((16,), jnp.float32)

    def zbody(j, carry):
        hist[pl.ds(j * 16, 16)] = zero16
        return carry

    lax.fori_loop(0, N // 16, zbody, 0)
    one16 = jnp.ones((16,), jnp.float32)

    def body(j, carry):
        idx = didx[pl.ds(j * 16, 16)]
        plsc.addupdate_scatter(hist, [idx], one16)
        return carry

    lax.fori_loop(0, EPW // 16, body, 0)
    pltpu.sync_copy(hist, out_hbm.at[wid])


NBUF = 5
GRP = NCHUNK // NBUF
NSP = 2


@functools.cache
def _build_agg_kernel():
    return pl.kernel(
        _agg_body,
        out_type=jax.ShapeDtypeStruct((NC, NS, NPT, H), jnp.float32),
        mesh=_sc_mesh(),
        scratch_types=[
            pltpu.VMEM((EPW,), jnp.int32),
            pltpu.VMEM((EPW,), jnp.int32),
            pltpu.VMEM((NBUF, CH), jnp.int32),
            pltpu.VMEM((NBUF, CH, H), jnp.float32),
            pltpu.VMEM_SHARED((N, H), jnp.float32),
            pltpu.VMEM_SHARED((N, H), jnp.float32),
            pltpu.SemaphoreType.DMA((NBUF,)),
            pltpu.SemaphoreType.DMA((NBUF,)),
        ],
        compiler_params=pltpu.CompilerParams(needs_layout_passes=False,
                                             use_tc_tiling_on_sc=False),
    )


def _agg_body(hs_hbm, src_hbm, dst_hbm, zeros_hbm, out_hbm,
              sall, dall, d80s, rowss, hsbuf, acc, gsems, ssems):
    c = lax.axis_index("c")
    s = lax.axis_index("s")
    wid = c * NS + s
    base = wid * EPW
    pltpu.sync_copy(src_hbm.at[pl.ds(base, EPW)], sall)
    pltpu.sync_copy(dst_hbm.at[pl.ds(base, EPW)], dall)
    pltpu.sync_copy(zeros_hbm.at[s], acc.at[pl.ds(s * NPT, NPT)])
    pltpu.sync_copy(hs_hbm.at[pl.ds(s * NPT, NPT)],
                    hsbuf.at[pl.ds(s * NPT, NPT)])
    plsc.subcore_barrier()

    def outer(g, carry):
        gathers = []
        for b in range(NBUF):
            @pl.when(g > 0)
            def _(b=b):
                pltpu.make_async_copy(rowss.at[b], acc.at[d80s.at[b]],
                                      ssems.at[b]).wait()

            off = (g * NBUF + b) * CH

            def cp(j, c2, b=b, off=off):
                d80s[b, pl.ds(j * 16, 16)] = dall[pl.ds(off + j * 16, 16)]
                return c2

            lax.fori_loop(0, CH // 16, cp, 0)
            src_ref = hsbuf if b >= NBUF - NSP else hs_hbm
            gathers.append(pltpu.async_copy(
                src_ref.at[sall.at[pl.ds(off, CH)]], rowss.at[b],
                gsems.at[b]))
        for b in range(NBUF):
            gathers[b].wait()
            pltpu.async_copy(rowss.at[b], acc.at[d80s.at[b]], ssems.at[b],
                             add=True)
        return carry

    lax.fori_loop(0, GRP, outer, 0)
    for b in range(NBUF):
        pltpu.make_async_copy(rowss.at[b], acc.at[d80s.at[b]],
                              ssems.at[b]).wait()
    plsc.subcore_barrier()
    pltpu.sync_copy(acc.at[pl.ds(s * NPT, NPT)], out_hbm.at[c, s])


def _tc1a_body(x_ref, p_ref, w0_ref, h0_ref):
    xp = (x_ref[...].reshape(B, R, D) + p_ref[...][None]).reshape(N, D)
    h0_ref[...] = lax.dot_general(xp, w0_ref[...], (((1,), (0,)), ((), ())),
                                  preferred_element_type=jnp.float32)


def _tc1a(x, prompt, W0):
    return pl.pallas_call(
        _tc1a_body,
        out_shape=jax.ShapeDtypeStruct((N, H), jnp.float32),
    )(x, prompt, W0)


def _tc1b_body(h0_ref, hist_ref, hs0_ref, dinv_ref):
    ones = jnp.ones((NW, 1), jnp.float32)
    deg = 1.0 + lax.dot_general(hist_ref[...], ones,
                                (((0,), (0,)), ((), ())),
                                precision=_HIGHEST,
                                preferred_element_type=jnp.float32)
    dinv = lax.rsqrt(deg)
    hs0_ref[...] = h0_ref[...] * dinv
    dinv_ref[...] = dinv


def _tc1b(h0, hist):
    return pl.pallas_call(
        _tc1b_body,
        out_shape=[
            jax.ShapeDtypeStruct((N, H), jnp.float32),
            jax.ShapeDtypeStruct((N, 1), jnp.float32),
        ],
    )(h0, hist)


def _conv_finish(aggp, h, dinv, b, g, be):
    s = aggp[0] + aggp[1]
    c1 = dinv * s + (dinv * dinv) * h + b[None, :]
    m = jnp.mean(c1, axis=0, keepdims=True)
    v = jnp.mean((c1 - m) * (c1 - m), axis=0, keepdims=True)
    return jnp.maximum(g[None, :] * (c1 - m) * lax.rsqrt(v + 1e-5)
                       + be[None, :], 0.0)


def _tc2_body(agg_ref, h0_ref, dinv_ref, b0_ref, g0_ref, be0_ref, w1_ref,
              h1_ref, hs1_ref):
    dinv = dinv_ref[...]
    r = _conv_finish(agg_ref[...], h0_ref[...], dinv,
                     b0_ref[...], g0_ref[...], be0_ref[...])
    h1 = lax.dot_general(r, w1_ref[...], (((1,), (0,)), ((), ())),
                         preferred_element_type=jnp.float32)
    h1_ref[...] = h1
    hs1_ref[...] = h1 * dinv


def _tc2(agg, h0, dinv, b0, g0, be0, W1):
    return pl.pallas_call(
        _tc2_body,
        out_shape=[
            jax.ShapeDtypeStruct((N, H), jnp.float32),
            jax.ShapeDtypeStruct((N, H), jnp.float32),
        ],
    )(agg, h0, dinv, b0, g0, be0, W1)


def _tc3_body(agg_ref, h1_ref, dinv_ref, b1_ref, g1_ref, be1_ref,
              brow_ref, bcol_ref,
              cw1_ref, cb1_ref, cw2_ref, cb2_ref, cw3_ref, cb3_ref,
              out_ref, ma_ref, mb_ref):
    hf = _conv_finish(agg_ref[...], h1_ref[...], dinv_ref[...],
                      b1_ref[...], g1_ref[...], be1_ref[...])
    bcol = bcol_ref[...]
    ma_ref[...] = hf
    bufs = [ma_ref, mb_ref]
    s = 1
    k = 0
    while s < N:
        src = bufs[k % 2]
        dst = bufs[(k + 1) % 2]
        same = bcol[s:] == bcol[:N - s]
        lo = src[pl.ds(0, N - s)]
        dst[pl.ds(0, N - s)] = jnp.where(
            same, jnp.maximum(lo, src[pl.ds(s, N - s)]), lo)
        dst[pl.ds(N - s, s)] = src[pl.ds(N - s, s)]
        s *= 2
        k += 1
    m = bufs[k % 2][...]
    bprev = jnp.concatenate(
        [jnp.full((1, 1), -1, jnp.int32), bcol[:-1]], axis=0)
    mb_ref[...] = jnp.where(bcol != bprev, m, 0.0)
    brow = brow_ref[...]
    giota = lax.broadcasted_iota(jnp.int32, (B, N), 0)
    maskr = (brow == giota).astype(jnp.float32)
    xsum = lax.dot_general(maskr, hf, (((1,), (0,)), ((), ())),
                           precision=_HIGHEST,
                           preferred_element_type=jnp.float32)
    xmax = lax.dot_general(maskr, mb_ref[...], (((1,), (0,)), ((), ())),
                           precision=_HIGHEST,
                           preferred_element_type=jnp.float32)
    cnt = lax.dot_general(maskr, jnp.ones((N, 1), jnp.float32),
                          (((1,), (0,)), ((), ())),
                          precision=_HIGHEST,
                          preferred_element_type=jnp.float32)
    xmean = xsum / jnp.maximum(cnt, 1.0)
    z = jnp.concatenate([xmean, xmax], axis=1)
    z = jnp.maximum(lax.dot_general(z, cw1_ref[...], (((1,), (0,)), ((), ())),
                                    preferred_element_type=jnp.float32)
                    + cb1_ref[...][None, :], 0.0)
    z = jnp.maximum(lax.dot_general(z, cw2_ref[...], (((1,), (0,)), ((), ())),
                                    preferred_element_type=jnp.float32)
                    + cb2_ref[...][None, :], 0.0)
    out_ref[...] = (lax.dot_general(z, cw3_ref[...], (((1,), (0,)), ((), ())),
                                    preferred_element_type=jnp.float32)
                    + cb3_ref[...][None, :])


def _tc3(agg, h1, dinv, b1, g1, be1, brow, bcol, cW1, cb1, cW2, cb2, cW3, cb3):
    return pl.pallas_call(
        _tc3_body,
        out_shape=jax.ShapeDtypeStruct((B, OUT), jnp.float32),
        scratch_shapes=[
            pltpu.VMEM((N, H), jnp.float32),
            pltpu.VMEM((N, H), jnp.float32),
        ],
    )(agg, h1, dinv, b1, g1, be1, brow, bcol,
      cW1, cb1, cW2, cb2, cW3, cb3)


def kernel(x, edge_index, batch, node_prompt, W0, b0, g0, be0,
           W1, b1, g1, be1, cW1, cb1, cW2, cb2, cW3, cb3):
    src_flat = edge_index[0]
    dst_flat = edge_index[1]
    prompt = node_prompt.reshape(R, D)
    brow = batch.reshape(1, N)
    bcol = batch.reshape(N, 1)
    zeros = jnp.zeros((NS, NPT, H), jnp.float32)

    deg_kernel = _build_deg_kernel()
    agg_kernel = _build_agg_kernel()
    hist = deg_kernel(dst_flat)
    h0 = _tc1a(x, prompt, W0)
    hs0, dinv = _tc1b(h0, hist)
    agg0 = agg_kernel(hs0, src_flat, dst_flat, zeros).reshape(NC, N, H)
    h1, hs1 = _tc2(agg0, h0, dinv, b0, g0, be0, W1)
    agg1 = agg_kernel(hs1, src_flat, dst_flat, zeros).reshape(NC, N, H)
    return _tc3(agg1, h1, dinv, b1, g1, be1, brow, bcol,
                cW1, cb1, cW2, cb2, cW3, cb3)


# speedup vs baseline: 1.3468x; 1.3468x over previous
import functools

import jax
import jax.numpy as jnp
from jax import lax
from jax.experimental import pallas as pl
from jax.experimental.pallas import tpu as pltpu
from jax.experimental.pallas import tpu_sc as plsc

N = 10000
E = 320000
D = 128
H = 64
R = 100
B = 100
OUT = 2

NC = 2
NS = 16
NW = NC * NS
EPW = E // NW
CH = 80
NCHUNK = EPW // CH
EROWS = E // CH
NPT = N // NS

_HIGHEST = lax.Precision.HIGHEST


def _sc_mesh():
    return plsc.VectorSubcoreMesh(core_axis_name="c", subcore_axis_name="s",
                                  num_cores=NC, num_subcores=NS)


@functools.cache
def _build_deg_kernel():
    return pl.kernel(
        _deg_body,
        out_type=jax.ShapeDtypeStruct((NW, N), jnp.float32),
        mesh=_sc_mesh(),
        scratch_types=[
            pltpu.VMEM((EPW,), jnp.int32),
            pltpu.VMEM((N,), jnp.float32),
        ],
        compiler_params=pltpu.CompilerParams(needs_layout_passes=False),
    )


def _deg_body(dst_hbm, out_hbm, didx, hist):
    c = lax.axis_index("c")
    s = lax.axis_index("s")
    wid = c * NS + s
    pltpu.sync_copy(dst_hbm.at[pl.ds(wid * EPW, EPW)], didx)
    zero16 = jnp.zeros((16,), jnp.float32)

    def zbody(j, carry):
        hist[pl.ds(j * 16, 16)] = zero16
        return carry

    lax.fori_loop(0, N // 16, zbody, 0)
    one16 = jnp.ones((16,), jnp.float32)

    def body(j, carry):
        idx = didx[pl.ds(j * 16, 16)]
        plsc.addupdate_scatter(hist, [idx], one16)
        return carry

    lax.fori_loop(0, EPW // 16, body, 0)
    pltpu.sync_copy(hist, out_hbm.at[wid])


NBUF = 5
GRP = NCHUNK // NBUF


@functools.cache
def _build_agg_kernel():
    return pl.kernel(
        _agg_body,
        out_type=jax.ShapeDtypeStruct((NC, NS, NPT, H), jnp.bfloat16),
        mesh=_sc_mesh(),
        scratch_types=[
            pltpu.VMEM((EPW,), jnp.int32),
            pltpu.VMEM((EPW,), jnp.int32),
            pltpu.VMEM((NBUF, CH), jnp.int32),
            pltpu.VMEM((NBUF, CH, H), jnp.bfloat16),
            pltpu.VMEM_SHARED((N, H), jnp.bfloat16),
            pltpu.SemaphoreType.DMA((NBUF,)),
            pltpu.SemaphoreType.DMA((NBUF,)),
        ],
        compiler_params=pltpu.CompilerParams(needs_layout_passes=False,
                                             use_tc_tiling_on_sc=False),
    )


def _agg_body(hs_hbm, src_hbm, dst_hbm, zeros_hbm, out_hbm,
              sall, dall, d80s, rowss, acc, gsems, ssems):
    c = lax.axis_index("c")
    s = lax.axis_index("s")
    wid = c * NS + s
    base = wid * EPW
    pltpu.sync_copy(src_hbm.at[pl.ds(base, EPW)], sall)
    pltpu.sync_copy(dst_hbm.at[pl.ds(base, EPW)], dall)
    pltpu.sync_copy(zeros_hbm.at[s], acc.at[pl.ds(s * NPT, NPT)])
    plsc.subcore_barrier()

    def outer(g, carry):
        gathers = []
        for b in range(NBUF):
            @pl.when(g > 0)
            def _(b=b):
                pltpu.make_async_copy(rowss.at[b], acc.at[d80s.at[b]],
                                      ssems.at[b]).wait()

            off = (g * NBUF + b) * CH

            def cp(j, c2, b=b, off=off):
                d80s[b, pl.ds(j * 16, 16)] = dall[pl.ds(off + j * 16, 16)]
                return c2

            lax.fori_loop(0, CH // 16, cp, 0)
            gathers.append(pltpu.async_copy(
                hs_hbm.at[sall.at[pl.ds(off, CH)]], rowss.at[b],
                gsems.at[b]))
        for b in range(NBUF):
            gathers[b].wait()
            pltpu.async_copy(rowss.at[b], acc.at[d80s.at[b]], ssems.at[b],
                             add=True)
        return carry

    lax.fori_loop(0, GRP, outer, 0)
    for b in range(NBUF):
        pltpu.make_async_copy(rowss.at[b], acc.at[d80s.at[b]],
                              ssems.at[b]).wait()
    plsc.subcore_barrier()
    pltpu.sync_copy(acc.at[pl.ds(s * NPT, NPT)], out_hbm.at[c, s])


def _tc1a_body(x_ref, p_ref, w0_ref, h0_ref):
    xp = (x_ref[...].reshape(B, R, D) + p_ref[...][None]).reshape(N, D)
    h0_ref[...] = lax.dot_general(xp, w0_ref[...], (((1,), (0,)), ((), ())),
                                  preferred_element_type=jnp.float32)


def _tc1a(x, prompt, W0):
    return pl.pallas_call(
        _tc1a_body,
        out_shape=jax.ShapeDtypeStruct((N, H), jnp.float32),
    )(x, prompt, W0)


def _tc1b_body(h0_ref, hist_ref, hs0_ref, dinv_ref):
    ones = jnp.ones((NW, 1), jnp.float32)
    deg = 1.0 + lax.dot_general(hist_ref[...], ones,
                                (((0,), (0,)), ((), ())),
                                precision=_HIGHEST,
                                preferred_element_type=jnp.float32)
    dinv = lax.rsqrt(deg)
    hs0_ref[...] = (h0_ref[...] * dinv).astype(jnp.bfloat16)
    dinv_ref[...] = dinv


def _tc1b(h0, hist):
    return pl.pallas_call(
        _tc1b_body,
        out_shape=[
            jax.ShapeDtypeStruct((N, H), jnp.bfloat16),
            jax.ShapeDtypeStruct((N, 1), jnp.float32),
        ],
    )(h0, hist)


def _conv_finish(aggp, h, dinv, b, g, be):
    s = (aggp[0].astype(jnp.float32) + aggp[1].astype(jnp.float32))
    c1 = dinv * s + (dinv * dinv) * h + b[None, :]
    m = jnp.mean(c1, axis=0, keepdims=True)
    v = jnp.mean((c1 - m) * (c1 - m), axis=0, keepdims=True)
    return jnp.maximum(g[None, :] * (c1 - m) * lax.rsqrt(v + 1e-5)
                       + be[None, :], 0.0)


def _tc2_body(agg_ref, h0_ref, dinv_ref, b0_ref, g0_ref, be0_ref, w1_ref,
              h1_ref, hs1_ref):
    dinv = dinv_ref[...]
    r = _conv_finish(agg_ref[...], h0_ref[...], dinv,
                     b0_ref[...], g0_ref[...], be0_ref[...])
    h1 = lax.dot_general(r, w1_ref[...], (((1,), (0,)), ((), ())),
                         preferred_element_type=jnp.float32)
    h1_ref[...] = h1
    hs1_ref[...] = (h1 * dinv).astype(jnp.bfloat16)


def _tc2(agg, h0, dinv, b0, g0, be0, W1):
    return pl.pallas_call(
        _tc2_body,
        out_shape=[
            jax.ShapeDtypeStruct((N, H), jnp.float32),
            jax.ShapeDtypeStruct((N, H), jnp.bfloat16),
        ],
    )(agg, h0, dinv, b0, g0, be0, W1)


def _tc3_body(agg_ref, h1_ref, dinv_ref, b1_ref, g1_ref, be1_ref,
              brow_ref, bcol_ref,
              cw1_ref, cb1_ref, cw2_ref, cb2_ref, cw3_ref, cb3_ref,
              out_ref, ma_ref, mb_ref):
    hf = _conv_finish(agg_ref[...], h1_ref[...], dinv_ref[...],
                      b1_ref[...], g1_ref[...], be1_ref[...])
    bcol = bcol_ref[...]
    ma_ref[...] = hf
    bufs = [ma_ref, mb_ref]
    s = 1
    k = 0
    while s < N:
        src = bufs[k % 2]
        dst = bufs[(k + 1) % 2]
        same = bcol[s:] == bcol[:N - s]
        lo = src[pl.ds(0, N - s)]
        dst[pl.ds(0, N - s)] = jnp.where(
            same, jnp.maximum(lo, src[pl.ds(s, N - s)]), lo)
        dst[pl.ds(N - s, s)] = src[pl.ds(N - s, s)]
        s *= 2
        k += 1
    m = bufs[k % 2][...]
    bprev = jnp.concatenate(
        [jnp.full((1, 1), -1, jnp.int32), bcol[:-1]], axis=0)
    mb_ref[...] = jnp.where(bcol != bprev, m, 0.0)
    brow = brow_ref[...]
    giota = lax.broadcasted_iota(jnp.int32, (B, N), 0)
    maskr = (brow == giota).astype(jnp.float32)
    xsum = lax.dot_general(maskr, hf, (((1,), (0,)), ((), ())),
                           precision=_HIGHEST,
                           preferred_element_type=jnp.float32)
    xmax = lax.dot_general(maskr, mb_ref[...], (((1,), (0,)), ((), ())),
                           precision=_HIGHEST,
                           preferred_element_type=jnp.float32)
    cnt = lax.dot_general(maskr, jnp.ones((N, 1), jnp.float32),
                          (((1,), (0,)), ((), ())),
                          precision=_HIGHEST,
                          preferred_element_type=jnp.float32)
    xmean = xsum / jnp.maximum(cnt, 1.0)
    z = jnp.concatenate([xmean, xmax], axis=1)
    z = jnp.maximum(lax.dot_general(z, cw1_ref[...], (((1,), (0,)), ((), ())),
                                    preferred_element_type=jnp.float32)
                    + cb1_ref[...][None, :], 0.0)
    z = jnp.maximum(lax.dot_general(z, cw2_ref[...], (((1,), (0,)), ((), ())),
                                    preferred_element_type=jnp.float32)
                    + cb2_ref[...][None, :], 0.0)
    out_ref[...] = (lax.dot_general(z, cw3_ref[...], (((1,), (0,)), ((), ())),
                                    preferred_element_type=jnp.float32)
                    + cb3_ref[...][None, :])


def _tc3(agg, h1, dinv, b1, g1, be1, brow, bcol, cW1, cb1, cW2, cb2, cW3, cb3):
    return pl.pallas_call(
        _tc3_body,
        out_shape=jax.ShapeDtypeStruct((B, OUT), jnp.float32),
        scratch_shapes=[
            pltpu.VMEM((N, H), jnp.float32),
            pltpu.VMEM((N, H), jnp.float32),
        ],
    )(agg, h1, dinv, b1, g1, be1, brow, bcol,
      cW1, cb1, cW2, cb2, cW3, cb3)


def kernel(x, edge_index, batch, node_prompt, W0, b0, g0, be0,
           W1, b1, g1, be1, cW1, cb1, cW2, cb2, cW3, cb3):
    src_flat = edge_index[0]
    dst_flat = edge_index[1]
    prompt = node_prompt.reshape(R, D)
    brow = batch.reshape(1, N)
    bcol = batch.reshape(N, 1)
    zeros = jnp.zeros((NS, NPT, H), jnp.bfloat16)

    deg_kernel = _build_deg_kernel()
    agg_kernel = _build_agg_kernel()
    hist = deg_kernel(dst_flat)
    h0 = _tc1a(x, prompt, W0)
    hs0, dinv = _tc1b(h0, hist)
    agg0 = agg_kernel(hs0, src_flat, dst_flat, zeros).reshape(NC, N, H)
    h1, hs1 = _tc2(agg0, h0, dinv, b0, g0, be0, W1)
    agg1 = agg_kernel(hs1, src_flat, dst_flat, zeros).reshape(NC, N, H)
    return _tc3(agg1, h1, dinv, b1, g1, be1, brow, bcol,
                cW1, cb1, cW2, cb2, cW3, cb3)
